# Initial kernel scaffold; baseline (speedup 1.0000x reference)
import jax
import jax.numpy as jnp
from jax.experimental import pallas as pl


def kernel(nodes, indicator, mask, Wg, bg, Wf, bf):
    raise NotImplementedError("write your pallas kernel here")


# speedup vs baseline: 4.4850x; 4.4850x over previous
import functools

import jax
import jax.numpy as jnp
from jax.experimental import pallas as pl
from jax.experimental.pallas import tpu as pltpu

N = 100000
D = 128
B = 64
R = 2000
NBLK = N // R


def _gated_readout_kernel(seg_ref, mask_ref, nodes_ref, wg_ref, bg_ref,
                          wf_ref, bf_ref, mean_ref, max_ref,
                          sum_acc, cnt_acc):
    i = pl.program_id(0)

    @pl.when(i == 0)
    def _init():
        sum_acc[...] = jnp.zeros_like(sum_acc)
        cnt_acc[...] = jnp.zeros_like(cnt_acc)
        max_ref[...] = jnp.full_like(max_ref, -jnp.inf)

    x = nodes_ref[...]
    g = jax.nn.sigmoid(
        jnp.dot(x, wg_ref[...], preferred_element_type=jnp.float32)
        + bg_ref[...])
    f = jnp.tanh(
        jnp.dot(x, wf_ref[...], preferred_element_type=jnp.float32)
        + bf_ref[...])
    mask_col = mask_ref[...]
    gated = g * f * mask_col

    seg_col = seg_ref[...]
    sidx = jax.lax.broadcasted_iota(jnp.int32, (R, B), 1)
    onehot = jnp.where(seg_col == sidx, mask_col, 0.0)

    dn = (((0,), (0,)), ((), ()))
    sum_acc[...] += jax.lax.dot_general(
        onehot, gated, dn, preferred_element_type=jnp.float32)
    cnt_acc[...] += jax.lax.dot_general(
        onehot, jnp.ones((R, 1), jnp.float32), dn,
        preferred_element_type=jnp.float32)

    s_lo = seg_col[0, 0]
    s_hi = seg_col[R - 1, 0]

    def body(s, _):
        vals = jnp.where(seg_col == s, gated, -jnp.inf)
        part = jnp.max(vals, axis=0, keepdims=True)
        cur = max_ref[pl.ds(s, 1), :]
        max_ref[pl.ds(s, 1), :] = jnp.maximum(cur, part)
        return 0

    jax.lax.fori_loop(s_lo, s_hi + 1, body, 0)

    @pl.when(i == NBLK - 1)
    def _final():
        mean_ref[...] = sum_acc[...] / jnp.maximum(cnt_acc[...], 1e-6)


@functools.partial(jax.jit, static_argnames=("interpret",))
def _run(nodes, indicator, mask, Wg, bg, Wf, bf, interpret=False):
    seg2 = indicator.astype(jnp.int32).reshape(N, 1)
    mask2 = mask.reshape(N, 1)
    bg2 = bg.reshape(1, D)
    bf2 = bf.reshape(1, D)

    mean, mx = pl.pallas_call(
        _gated_readout_kernel,
        grid=(NBLK,),
        in_specs=[
            pl.BlockSpec((R, 1), lambda i: (i, 0)),
            pl.BlockSpec((R, 1), lambda i: (i, 0)),
            pl.BlockSpec((R, D), lambda i: (i, 0)),
            pl.BlockSpec((D, D), lambda i: (0, 0)),
            pl.BlockSpec((1, D), lambda i: (0, 0)),
            pl.BlockSpec((D, D), lambda i: (0, 0)),
            pl.BlockSpec((1, D), lambda i: (0, 0)),
        ],
        out_specs=[
            pl.BlockSpec((B, D), lambda i: (0, 0)),
            pl.BlockSpec((B, D), lambda i: (0, 0)),
        ],
        out_shape=[
            jax.ShapeDtypeStruct((B, D), jnp.float32),
            jax.ShapeDtypeStruct((B, D), jnp.float32),
        ],
        scratch_shapes=[
            pltpu.VMEM((B, D), jnp.float32),
            pltpu.VMEM((B, 1), jnp.float32),
        ],
        compiler_params=pltpu.CompilerParams(
            dimension_semantics=("arbitrary",),
        ),
        interpret=interpret,
    )(seg2, mask2, nodes, Wg, bg2, Wf, bf2)
    return jnp.concatenate([mean, mx], axis=-1)


def kernel(nodes, indicator, mask, Wg, bg, Wf, bf):
    return _run(nodes, indicator, mask, Wg, bg, Wf, bf)


# speedup vs baseline: 4.5117x; 1.0059x over previous
import functools

import jax
import jax.numpy as jnp
from jax.experimental import pallas as pl
from jax.experimental.pallas import tpu as pltpu

N = 100000
D = 128
B = 64
R = 2000
NBLK = N // R


def _gated_readout_kernel(segr_ref, mask_ref, seg_ref, nodes_ref, w_ref,
                          b_ref, mean_ref, max_ref, sum_acc, cnt_acc):
    i = pl.program_id(0)

    @pl.when(i == 0)
    def _init():
        sum_acc[...] = jnp.zeros_like(sum_acc)
        cnt_acc[...] = jnp.zeros_like(cnt_acc)
        max_ref[...] = jnp.full_like(max_ref, -jnp.inf)

    x = nodes_ref[...]
    xw = jnp.dot(x, w_ref[...], preferred_element_type=jnp.float32)
    xw = xw + b_ref[...]
    g = jax.nn.sigmoid(xw[:, :D])
    f = jnp.tanh(xw[:, D:])
    seg_row = segr_ref[0]
    mask_col = mask_ref[...]
    gated = g * f * mask_col

    bidx = jax.lax.broadcasted_iota(jnp.int32, (B, R), 0)
    onehot_t = jnp.where(seg_row == bidx, 1.0, 0.0)

    sum_acc[...] += jnp.dot(onehot_t, gated,
                            preferred_element_type=jnp.float32)
    cnt_acc[...] += jnp.dot(onehot_t, mask_col,
                            preferred_element_type=jnp.float32)

    seg_col = seg_ref[...]
    s_lo = seg_col[0, 0]
    s_hi = seg_col[R - 1, 0]

    def body(s, _):
        vals = jnp.where(seg_col == s, gated, -jnp.inf)
        part = jnp.max(vals, axis=0, keepdims=True)
        cur = max_ref[pl.ds(s, 1), :]
        max_ref[pl.ds(s, 1), :] = jnp.maximum(cur, part)
        return 0

    jax.lax.fori_loop(s_lo, s_hi + 1, body, 0)

    @pl.when(i == NBLK - 1)
    def _final():
        mean_ref[...] = sum_acc[...] / jnp.maximum(cnt_acc[...], 1e-6)


@functools.partial(jax.jit, static_argnames=("interpret",))
def _run(nodes, indicator, mask, Wg, bg, Wf, bf, interpret=False):
    seg = indicator.astype(jnp.int32)
    seg3 = seg.reshape(NBLK, 1, R)
    mask2 = mask.reshape(N, 1)
    seg2 = seg.reshape(N, 1)
    w2 = jnp.concatenate([Wg, Wf], axis=1)
    b2 = jnp.concatenate([bg, bf]).reshape(1, 2 * D)

    mean, mx = pl.pallas_call(
        _gated_readout_kernel,
        grid=(NBLK,),
        in_specs=[
            pl.BlockSpec((1, 1, R), lambda i: (i, 0, 0)),
            pl.BlockSpec((R, 1), lambda i: (i, 0)),
            pl.BlockSpec((R, 1), lambda i: (i, 0)),
            pl.BlockSpec((R, D), lambda i: (i, 0)),
            pl.BlockSpec((D, 2 * D), lambda i: (0, 0)),
            pl.BlockSpec((1, 2 * D), lambda i: (0, 0)),
        ],
        out_specs=[
            pl.BlockSpec((B, D), lambda i: (0, 0)),
            pl.BlockSpec((B, D), lambda i: (0, 0)),
        ],
        out_shape=[
            jax.ShapeDtypeStruct((B, D), jnp.float32),
            jax.ShapeDtypeStruct((B, D), jnp.float32),
        ],
        scratch_shapes=[
            pltpu.VMEM((B, D), jnp.float32),
            pltpu.VMEM((B, 1), jnp.float32),
        ],
        compiler_params=pltpu.CompilerParams(
            dimension_semantics=("arbitrary",),
        ),
        interpret=interpret,
    )(seg3, mask2, seg2, nodes, w2, b2)
    return jnp.concatenate([mean, mx], axis=-1)


def kernel(nodes, indicator, mask, Wg, bg, Wf, bf):
    return _run(nodes, indicator, mask, Wg, bg, Wf, bf)
